# baseline (speedup 1.0000x reference)
import jax
import jax.numpy as jnp
from jax import lax
from jax.experimental import pallas as pl
from jax.experimental.pallas import tpu as pltpu

N = 16384
B = 256
NP = 64
D = 128
K = 8
CELLS_PER_PROG = 64
PTS = CELLS_PER_PROG * NP
GRID = B // CELLS_PER_PROG

_f32 = jnp.float32
_bf16 = jnp.bfloat16
_i32 = jnp.int32


def _l2n_rows(x):
    n = jnp.sqrt(jnp.sum(x * x, axis=-1, keepdims=True))
    return x / jnp.maximum(n, 1e-12)


def _body(cp_ref, cls_ref, tbl_ref,
          cw1_ref, cb1_ref, cw2_ref, cb2_ref,
          pw1_ref, pb1_ref, pw2_ref, pb2_ref,
          mw_ref, mb_ref,
          ew1t_ref, ew1b_ref, ab_ref,
          ew2_ref, eb2_ref,
          lw1_ref, lb1_ref, lw2_ref, lb2_ref,
          out_ref):
    tbln = _l2n_rows(tbl_ref[:, :]).astype(_bf16)
    ci = cls_ref[0]
    oh_t = (ci == lax.broadcasted_iota(_i32, (NP, PTS), 0)).astype(_bf16)
    ce = lax.dot_general(oh_t, tbln, (((0,), (0,)), ((), ())),
                         preferred_element_type=_f32)

    cp = cp_ref[:, :]
    colh = jnp.maximum(
        jnp.dot(cp, cw1_ref[:, :], preferred_element_type=_f32)
        + cb1_ref[0], 0.0)
    col = _l2n_rows(jnp.dot(colh, cw2_ref[:, :], preferred_element_type=_f32)
                    + cb2_ref[0])
    posh = jnp.maximum(
        jnp.dot(cp, pw1_ref[:, :], preferred_element_type=_f32)
        + pb1_ref[0], 0.0)
    pos = _l2n_rows(jnp.dot(posh, pw2_ref[:, :], preferred_element_type=_f32)
                    + pb2_ref[0])

    cat = jnp.concatenate([ce, col, pos], axis=1).astype(_bf16)
    emb = (jnp.dot(cat, mw_ref[:, :], preferred_element_type=_f32)
           + mb_ref[0])

    emb_bf = emb.astype(_bf16)
    emb_lo = (emb - emb_bf.astype(_f32)).astype(_bf16)
    emb_hl = jnp.concatenate([emb_bf, emb_lo], axis=1)

    a_all = (jnp.dot(emb_bf, ew1t_ref[:, :], preferred_element_type=_f32)
             + ab_ref[0])

    eye = (lax.broadcasted_iota(_i32, (NP, NP), 0)
           == lax.broadcasted_iota(_i32, (NP, NP), 1)).astype(_bf16)
    eye_rep = jnp.concatenate([eye] * K, axis=1)
    kmax = _i32(0x7FFFFFFF)

    GC = 16
    GPTS = GC * NP
    iota_s = lax.broadcasted_iota(_i32, (NP, GPTS), 0)

    diffs, areps = [], []
    for g in range(CELLS_PER_PROG // GC):
        cells = range(g * GC, (g + 1) * GC)
        dcs = []
        for c in cells:
            xs = emb_bf[c * NP:(c + 1) * NP, :]
            gram = lax.dot_general(xs, xs, (((1,), (1,)), ((), ())),
                                   preferred_element_type=_f32)
            xf = emb[c * NP:(c + 1) * NP, :]
            sq = jnp.sum(xf * xf, axis=1, keepdims=True)
            dcs.append(sq - 2.0 * gram)
        d = jnp.concatenate(dcs, axis=1)

        ui = lax.bitcast_convert_type(d, _i32)
        key = (ui ^ ((ui >> 31) & _i32(0x7FFFFFFF)))
        key = (key & _i32(-64)) | iota_s
        oh_list = []
        for _ in range(K):
            mk = jnp.min(key, axis=0, keepdims=True)
            ohk = key == mk
            oh_list.append(ohk.astype(_bf16))
            key = jnp.where(ohk, kmax, key)

        for ci, c in enumerate(cells):
            lo, hi_ = c * NP, (c + 1) * NP
            glo, ghi = ci * NP, (ci + 1) * NP
            m_c = (jnp.concatenate(
                [oh_list[k][:, glo:ghi] for k in range(K)], axis=1)
                - eye_rep)
            hl = lax.dot_general(m_c, emb_hl[lo:hi_],
                                 (((0,), (0,)), ((), ())),
                                 preferred_element_type=_f32)
            diffs.append((hl[:, :D] + hl[:, D:]).astype(_bf16))
            areps.append(jnp.concatenate([a_all[lo:hi_]] * K, axis=0))

    diff = jnp.concatenate(diffs, axis=0)
    h = jnp.concatenate(areps, axis=0) + jnp.dot(
        diff, ew1b_ref[:, :], preferred_element_type=_f32)
    h = jnp.maximum(h.astype(_bf16), _bf16(0.0))
    y = jnp.dot(h, ew2_ref[:, :], preferred_element_type=_f32)

    gcs = [jnp.max(y[c * K * NP:(c + 1) * K * NP], axis=0, keepdims=True)
           for c in range(CELLS_PER_PROG)]
    g = jnp.concatenate(gcs, axis=0) + eb2_ref[0]
    o = jnp.maximum(jnp.dot(g, lw1_ref[:, :], preferred_element_type=_f32)
                    + lb1_ref[0], 0.0)
    o = jnp.dot(o, lw2_ref[:, :], preferred_element_type=_f32) + lb2_ref[0]
    out_ref[:, :] = _l2n_rows(o)


def kernel(class_indices, colors, positions, batch, class_table,
           pw1, pb1, pw2, pb2, cw1, cb1, cw2, cb2, mw, mb,
           ew1, eb1, bng, bnb, ew2, eb2, lw1, lb1, lw2, lb2):
    del batch
    f32 = _f32
    cp = jnp.pad(jnp.concatenate([colors.astype(f32), positions.astype(f32)],
                                 axis=1), ((0, 0), (0, 2)))
    cls3 = class_indices.reshape(GRID, 1, PTS)
    tbl_p = jnp.pad(class_table.astype(f32),
                    ((0, NP - class_table.shape[0]), (0, 0)))
    cw1p = jnp.pad(cw1.astype(f32), ((0, 5), (0, 0)))
    pw1p = jnp.pad(pw1.astype(f32), ((3, 2), (0, 0)))
    s = bng.astype(f32) / jnp.sqrt(jnp.asarray(1.0 + 1e-5, f32))
    ew1t_s = ew1[:D].astype(f32) * s[None, :]
    ew1b_s = ew1[D:].astype(f32) * s[None, :]
    ab = (eb1.astype(f32) * s + bnb.astype(f32)).reshape(1, D)

    r2 = lambda v: v.reshape(1, -1).astype(f32)
    bf = lambda v: v.astype(_bf16)

    grid_spec = pl.GridSpec(
        grid=(GRID,),
        in_specs=[
            pl.BlockSpec((PTS, 8), lambda i: (i, 0)),
            pl.BlockSpec((1, 1, PTS), lambda i: (i, 0, 0)),
            pl.BlockSpec((NP, D), lambda i: (0, 0)),
            pl.BlockSpec((8, 64), lambda i: (0, 0)),
            pl.BlockSpec((1, 64), lambda i: (0, 0)),
            pl.BlockSpec((64, D), lambda i: (0, 0)),
            pl.BlockSpec((1, D), lambda i: (0, 0)),
            pl.BlockSpec((8, 64), lambda i: (0, 0)),
            pl.BlockSpec((1, 64), lambda i: (0, 0)),
            pl.BlockSpec((64, D), lambda i: (0, 0)),
            pl.BlockSpec((1, D), lambda i: (0, 0)),
            pl.BlockSpec((3 * D, D), lambda i: (0, 0)),
            pl.BlockSpec((1, D), lambda i: (0, 0)),
            pl.BlockSpec((D, D), lambda i: (0, 0)),
            pl.BlockSpec((D, D), lambda i: (0, 0)),
            pl.BlockSpec((1, D), lambda i: (0, 0)),
            pl.BlockSpec((D, D), lambda i: (0, 0)),
            pl.BlockSpec((1, D), lambda i: (0, 0)),
            pl.BlockSpec((D, D), lambda i: (0, 0)),
            pl.BlockSpec((1, D), lambda i: (0, 0)),
            pl.BlockSpec((D, D), lambda i: (0, 0)),
            pl.BlockSpec((1, D), lambda i: (0, 0)),
        ],
        out_specs=pl.BlockSpec((CELLS_PER_PROG, D), lambda i: (i, 0)),
    )
    return pl.pallas_call(
        _body,
        grid_spec=grid_spec,
        out_shape=jax.ShapeDtypeStruct((B, D), f32),
        compiler_params=pltpu.CompilerParams(
            dimension_semantics=("parallel",)),
    )(cp, cls3, tbl_p,
      cw1p, r2(cb1), cw2.astype(f32), r2(cb2),
      pw1p, r2(pb1), pw2.astype(f32), r2(pb2),
      bf(mw), r2(mb),
      bf(ew1t_s), bf(ew1b_s), ab,
      bf(ew2), r2(eb2),
      lw1.astype(f32), r2(lb1), lw2.astype(f32), r2(lb2))


# speedup vs baseline: 1.0301x; 1.0301x over previous
import jax
import jax.numpy as jnp
from jax import lax
from jax.experimental import pallas as pl
from jax.experimental.pallas import tpu as pltpu

N = 16384
NC = 33
B = 256
NP = 64
D = 128
K = 8
CELLS_PER_PROG = 64
PTS = CELLS_PER_PROG * NP
GRID = B // CELLS_PER_PROG

_f32 = jnp.float32
_bf16 = jnp.bfloat16
_i32 = jnp.int32


def _l2n_rows(x):
    n = jnp.sqrt(jnp.sum(x * x, axis=-1, keepdims=True))
    return x / jnp.maximum(n, 1e-12)


def _body(col_ref, pos_ref, cls_ref, tbl_ref,
          cw1_ref, cb1_ref, cw2_ref, cb2_ref,
          pw1_ref, pb1_ref, pw2_ref, pb2_ref,
          mw_ref, mb_ref,
          ew1_ref, eb1_ref, bng_ref, bnb_ref,
          ew2_ref, eb2_ref,
          lw1_ref, lb1_ref, lw2_ref, lb2_ref,
          out_ref):
    tbln = jnp.concatenate(
        [_l2n_rows(tbl_ref[:, :]).astype(_bf16),
         jnp.zeros((NP - NC, D), _bf16)], axis=0)
    ci = cls_ref[0]
    oh_t = (ci == lax.broadcasted_iota(_i32, (NP, PTS), 0)).astype(_bf16)
    ce = lax.dot_general(oh_t, tbln, (((0,), (0,)), ((), ())),
                         preferred_element_type=_f32)

    colh = jnp.maximum(
        jnp.dot(col_ref[:, :], cw1_ref[:, :], preferred_element_type=_f32)
        + cb1_ref[0], 0.0)
    col = _l2n_rows(jnp.dot(colh, cw2_ref[:, :], preferred_element_type=_f32)
                    + cb2_ref[0])
    posh = jnp.maximum(
        jnp.dot(pos_ref[:, :], pw1_ref[:, :], preferred_element_type=_f32)
        + pb1_ref[0], 0.0)
    pos = _l2n_rows(jnp.dot(posh, pw2_ref[:, :], preferred_element_type=_f32)
                    + pb2_ref[0])

    cat = jnp.concatenate([ce, col, pos], axis=1).astype(_bf16)
    emb = (jnp.dot(cat, mw_ref[:, :].astype(_bf16), preferred_element_type=_f32)
           + mb_ref[0])

    emb_bf = emb.astype(_bf16)
    emb_lo = (emb - emb_bf.astype(_f32)).astype(_bf16)
    emb_hl = jnp.concatenate([emb_bf, emb_lo], axis=1)

    s = bng_ref[0] / jnp.sqrt(jnp.asarray(1.0 + 1e-5, _f32))
    ew1t_s = (ew1_ref[:D, :] * s).astype(_bf16)
    ew1b_s = (ew1_ref[D:, :] * s).astype(_bf16)
    ab = eb1_ref[0] * s + bnb_ref[0]
    ew2_bf = ew2_ref[:, :].astype(_bf16)
    a_all = (jnp.dot(emb_bf, ew1t_s, preferred_element_type=_f32)
             + ab)

    eye = (lax.broadcasted_iota(_i32, (NP, NP), 0)
           == lax.broadcasted_iota(_i32, (NP, NP), 1)).astype(_bf16)
    eye_rep = jnp.concatenate([eye] * K, axis=1)
    kmax = _i32(0x7FFFFFFF)

    GC = CELLS_PER_PROG
    GPTS = GC * NP
    iota_s = lax.broadcasted_iota(_i32, (NP, GPTS), 0)

    diffs, areps = [], []
    for g in range(CELLS_PER_PROG // GC):
        cells = range(g * GC, (g + 1) * GC)
        dcs = []
        for c in cells:
            xs = emb_bf[c * NP:(c + 1) * NP, :]
            gram = lax.dot_general(xs, xs, (((1,), (1,)), ((), ())),
                                   preferred_element_type=_f32)
            xf = emb[c * NP:(c + 1) * NP, :]
            sq = jnp.sum(xf * xf, axis=1, keepdims=True)
            dcs.append(sq - 2.0 * gram)
        d = jnp.concatenate(dcs, axis=1)

        ui = lax.bitcast_convert_type(d, _i32)
        key = (ui ^ ((ui >> 31) & _i32(0x7FFFFFFF)))
        key = (key & _i32(-64)) | iota_s
        oh_list = []
        for _ in range(K):
            mk = jnp.min(key, axis=0, keepdims=True)
            ohk = key == mk
            oh_list.append(ohk.astype(_bf16))
            key = jnp.where(ohk, kmax, key)

        for ci, c in enumerate(cells):
            lo, hi_ = c * NP, (c + 1) * NP
            glo, ghi = ci * NP, (ci + 1) * NP
            m_c = (jnp.concatenate(
                [oh_list[k][:, glo:ghi] for k in range(K)], axis=1)
                - eye_rep)
            hl = lax.dot_general(m_c, emb_hl[lo:hi_],
                                 (((0,), (0,)), ((), ())),
                                 preferred_element_type=_f32)
            diffs.append((hl[:, :D] + hl[:, D:]).astype(_bf16))
            areps.append(jnp.concatenate([a_all[lo:hi_]] * K, axis=0))

    diff = jnp.concatenate(diffs, axis=0)
    h = jnp.concatenate(areps, axis=0) + jnp.dot(
        diff, ew1b_s, preferred_element_type=_f32)
    h = jnp.maximum(h.astype(_bf16), _bf16(0.0))
    y = jnp.dot(h, ew2_bf, preferred_element_type=_f32)

    gcs = [jnp.max(y[c * K * NP:(c + 1) * K * NP], axis=0, keepdims=True)
           for c in range(CELLS_PER_PROG)]
    g = jnp.concatenate(gcs, axis=0) + eb2_ref[0]
    o = jnp.maximum(jnp.dot(g, lw1_ref[:, :], preferred_element_type=_f32)
                    + lb1_ref[0], 0.0)
    o = jnp.dot(o, lw2_ref[:, :], preferred_element_type=_f32) + lb2_ref[0]
    out_ref[:, :] = _l2n_rows(o)


def kernel(class_indices, colors, positions, batch, class_table,
           pw1, pb1, pw2, pb2, cw1, cb1, cw2, cb2, mw, mb,
           ew1, eb1, bng, bnb, ew2, eb2, lw1, lb1, lw2, lb2):
    del batch
    f32 = _f32
    cls3 = class_indices.reshape(GRID, 1, PTS)
    r2 = lambda v: v.reshape(1, -1).astype(f32)

    grid_spec = pl.GridSpec(
        grid=(GRID,),
        in_specs=[
            pl.BlockSpec((PTS, 3), lambda i: (i, 0)),
            pl.BlockSpec((PTS, 3), lambda i: (i, 0)),
            pl.BlockSpec((1, 1, PTS), lambda i: (i, 0, 0)),
            pl.BlockSpec((NC, D), lambda i: (0, 0)),
            pl.BlockSpec((3, 64), lambda i: (0, 0)),
            pl.BlockSpec((1, 64), lambda i: (0, 0)),
            pl.BlockSpec((64, D), lambda i: (0, 0)),
            pl.BlockSpec((1, D), lambda i: (0, 0)),
            pl.BlockSpec((3, 64), lambda i: (0, 0)),
            pl.BlockSpec((1, 64), lambda i: (0, 0)),
            pl.BlockSpec((64, D), lambda i: (0, 0)),
            pl.BlockSpec((1, D), lambda i: (0, 0)),
            pl.BlockSpec((3 * D, D), lambda i: (0, 0)),
            pl.BlockSpec((1, D), lambda i: (0, 0)),
            pl.BlockSpec((2 * D, D), lambda i: (0, 0)),
            pl.BlockSpec((1, D), lambda i: (0, 0)),
            pl.BlockSpec((1, D), lambda i: (0, 0)),
            pl.BlockSpec((1, D), lambda i: (0, 0)),
            pl.BlockSpec((D, D), lambda i: (0, 0)),
            pl.BlockSpec((1, D), lambda i: (0, 0)),
            pl.BlockSpec((D, D), lambda i: (0, 0)),
            pl.BlockSpec((1, D), lambda i: (0, 0)),
            pl.BlockSpec((D, D), lambda i: (0, 0)),
            pl.BlockSpec((1, D), lambda i: (0, 0)),
        ],
        out_specs=pl.BlockSpec((CELLS_PER_PROG, D), lambda i: (i, 0)),
    )
    return pl.pallas_call(
        _body,
        grid_spec=grid_spec,
        out_shape=jax.ShapeDtypeStruct((B, D), f32),
        compiler_params=pltpu.CompilerParams(
            dimension_semantics=("parallel",)),
    )(colors, positions, cls3, class_table,
      cw1, r2(cb1), cw2, r2(cb2),
      pw1, r2(pb1), pw2, r2(pb2),
      mw, r2(mb),
      ew1, r2(eb1), r2(bng), r2(bnb),
      ew2, r2(eb2),
      lw1, r2(lb1), lw2, r2(lb2))
